# baseline (speedup 1.0000x reference)
import jax
import jax.numpy as jnp
from jax.experimental import pallas as pl
from jax.experimental.pallas import tpu as pltpu


def _partial_sums(logits, target):
    pred = jax.nn.sigmoid(logits)
    pred = jnp.clip(pred, 1e-4, 1.0 - 1e-4)
    pos = (target >= 0.9999).astype(jnp.float32)
    omp = 1.0 - pred
    pos_loss = jnp.log(pred) * (omp * omp) * pos
    omt = 1.0 - target
    omt2 = omt * omt
    neg_loss = jnp.log(omp) * (pred * pred) * (omt2 * omt2) * (1.0 - pos)
    return jnp.sum(pos_loss + neg_loss), jnp.sum(pos)


def _fused_kernel(nblocks, mlog, mmask, vlog, vmask, out_ref, acc):
    i = pl.program_id(0)

    @pl.when(i == 0)
    def _init():
        acc[0] = 0.0
        acc[1] = 0.0
        acc[2] = 0.0
        acc[3] = 0.0

    sm, cm = _partial_sums(mlog[...], mmask[...])
    sv, cv = _partial_sums(vlog[...], vmask[...])
    acc[0] = acc[0] + sm
    acc[1] = acc[1] + cm
    acc[2] = acc[2] + sv
    acc[3] = acc[3] + cv

    @pl.when(i == nblocks - 1)
    def _finish():
        num_pos_m = jnp.maximum(acc[1], 1.0)
        num_pos_v = jnp.maximum(acc[3], 1.0)
        out_ref[0] = -(acc[0] / num_pos_m) - (acc[2] / num_pos_v)


def kernel(main_kf_logits, main_kf_mask, vertex_kf_logits, vertex_kf_mask):
    B, C, H, W = main_kf_logits.shape
    CV = vertex_kf_logits.shape[1]
    rows_m = B * C * H
    rows_v = B * CV * H
    mlog = main_kf_logits.reshape(rows_m, W)
    mmask = main_kf_mask.reshape(rows_m, W)
    vlog = vertex_kf_logits.reshape(rows_v, W)
    vmask = vertex_kf_mask.reshape(rows_v, W)

    nblocks = 16
    assert rows_m % nblocks == 0 and rows_v % nblocks == 0
    br_m = rows_m // nblocks
    br_v = rows_v // nblocks

    import functools
    out = pl.pallas_call(
        functools.partial(_fused_kernel, nblocks),
        grid=(nblocks,),
        in_specs=[
            pl.BlockSpec((br_m, W), lambda i: (i, 0)),
            pl.BlockSpec((br_m, W), lambda i: (i, 0)),
            pl.BlockSpec((br_v, W), lambda i: (i, 0)),
            pl.BlockSpec((br_v, W), lambda i: (i, 0)),
        ],
        out_specs=pl.BlockSpec(memory_space=pltpu.SMEM),
        out_shape=jax.ShapeDtypeStruct((1,), jnp.float32),
        scratch_shapes=[pltpu.SMEM((4,), jnp.float32)],
    )(mlog, mmask, vlog, vmask)
    return out[0]
